# baseline (speedup 1.0000x reference)
import jax
import jax.numpy as jnp
from jax import lax
from jax.experimental import pallas as pl
from jax.experimental.pallas import tpu as pltpu
from jax.experimental.pallas import tpu_sc as plsc

N_NODES = 10000
N_EDGES = 320000
D = 128
BN = 1000
CH = 80
NC, NS = 2, 16
NW = NC * NS
NCHUNK = N_EDGES // CH
CPW = NCHUNK // NW
SB = 5
NSB = CPW // SB
N_AGG = 10112
RPT = N_AGG // NS


def _s1_body(x_ref, vt_ref, wax_ref, wbx_ref, ne_ref, wae_ref, wbe_ref,
             bm_ref, ab_ref):
    oh = (vt_ref[...] ==
          lax.broadcasted_iota(jnp.int32, (1, 8), 1).astype(jnp.float32)
          ).astype(jnp.float32)
    ca = jnp.dot(ne_ref[...], wae_ref[...], preferred_element_type=jnp.float32)
    cb = jnp.dot(ne_ref[...], wbe_ref[...], preferred_element_type=jnp.float32)
    xb = x_ref[...]
    ab_ref[0] = (jnp.dot(xb, wax_ref[...], preferred_element_type=jnp.float32)
                 + jnp.dot(oh, ca, preferred_element_type=jnp.float32))
    ab_ref[1] = (jnp.dot(xb, wbx_ref[...], preferred_element_type=jnp.float32)
                 + jnp.dot(oh, cb, preferred_element_type=jnp.float32)
                 + bm_ref[...])


def _stage1(x, vtf, wax, wbx, ne8, wae16, wbe16, bm):
    row = pl.BlockSpec((BN, D), lambda i: (i, 0))
    full = lambda shape: pl.BlockSpec(shape, lambda i: tuple(0 for _ in shape))
    return pl.pallas_call(
        _s1_body,
        grid=(N_NODES // BN,),
        in_specs=[row, pl.BlockSpec((BN, 1), lambda i: (i, 0)),
                  full((D, D)), full((D, D)), full((8, 16)),
                  full((16, D)), full((16, D)), full((1, D))],
        out_specs=pl.BlockSpec((2, BN, D), lambda i: (0, i, 0)),
        out_shape=jax.ShapeDtypeStruct((2, N_NODES, D), jnp.float32),
    )(x, vtf, wax, wbx, ne8, wae16, wbe16, bm)


def _sc_edge_body(ab_hbm, idx_hbm, out_hbm,
                  idx_v, ab_v, agg_sh, sem_g, sem_b, sem_s, sem_i):
    c = lax.axis_index("c")
    s = lax.axis_index("s")
    w = s * NC + c
    row0 = s * RPT
    @plsc.parallel_loop(0, CH, unroll=8)
    def zero_body(r):
        for g in range(8):
            ab_v[0, r, pl.ds(g * 16, 16)] = jnp.zeros((16,), jnp.float32)

    for k in range(7):
        pltpu.sync_copy(ab_v.at[0, pl.ds(0, 80)],
                        agg_sh.at[pl.ds(row0 + k * 80, 80)])
    pltpu.sync_copy(ab_v.at[0, pl.ds(0, 72)],
                    agg_sh.at[pl.ds(row0 + 560, 72)])
    plsc.subcore_barrier()

    def drain_scatter(slot):
        pltpu.make_async_copy(ab_v.at[slot],
                              agg_sh.at[idx_v.at[0, 0, 2]],
                              sem_s.at[slot]).wait()

    def blk_j(i):
        return lax.rem(i // SB, 2), lax.rem(i, SB)

    def start_a(i):
        slot = lax.rem(i, 4)
        blk, j = blk_j(i)

        @pl.when(i >= 4)
        def _():
            drain_scatter(slot)

        pltpu.async_copy(ab_hbm.at[idx_v.at[blk, j, 0]], ab_v.at[slot],
                         sem_g.at[slot])

    def start_b(i):
        slot = lax.rem(i, 4)
        blk, j = blk_j(i)
        pltpu.make_async_copy(ab_hbm.at[idx_v.at[blk, j, 0]], ab_v.at[slot],
                              sem_g.at[slot]).wait()
        pltpu.async_copy(ab_hbm.at[idx_v.at[blk, j, 1]], ab_v.at[slot],
                         sem_b.at[slot], add=True)

    pltpu.sync_copy(idx_hbm.at[w, 0], idx_v.at[0])
    start_a(0)
    start_a(1)
    start_b(0)

    def chunk_body(i, carry):
        slot = lax.rem(i, 4)
        blk, j = blk_j(i)

        @pl.when((j == SB - 3) & (i + 3 < CPW))
        def _():
            pltpu.async_copy(idx_hbm.at[w, i // SB + 1],
                             idx_v.at[1 - blk], sem_i)

        @pl.when((j == SB - 2) & (i + 2 < CPW))
        def _():
            pltpu.make_async_copy(idx_hbm.at[w, 0], idx_v.at[1 - blk],
                                  sem_i).wait()

        @pl.when(i + 2 < CPW)
        def _():
            start_a(i + 2)

        @pl.when(i + 1 < CPW)
        def _():
            start_b(i + 1)

        pltpu.make_async_copy(ab_hbm.at[idx_v.at[blk, j, 1]], ab_v.at[slot],
                              sem_b.at[slot]).wait()

        @plsc.parallel_loop(0, CH, unroll=16)
        def row_body(r):
            for g in range(8):
                sl = pl.ds(g * 16, 16)
                ab_v[slot, r, sl] = jnp.maximum(ab_v[slot, r, sl], 0.0)

        pltpu.async_copy(ab_v.at[slot], agg_sh.at[idx_v.at[blk, j, 2]],
                         sem_s.at[slot], add=True)
        return carry

    lax.fori_loop(0, CPW, chunk_body, 0)
    drain_scatter(0)
    drain_scatter(1)
    drain_scatter(2)
    drain_scatter(3)
    plsc.subcore_barrier()
    pltpu.sync_copy(agg_sh.at[pl.ds(row0, RPT)],
                    out_hbm.at[c, pl.ds(row0, RPT)])


def _stage2(ab_flat, idx3):
    mesh = plsc.VectorSubcoreMesh(core_axis_name="c", subcore_axis_name="s")
    k = pl.kernel(
        _sc_edge_body,
        out_type=jax.ShapeDtypeStruct((NC, N_AGG, D), jnp.float32),
        mesh=mesh,
        scratch_types=[
            pltpu.VMEM((2, SB, 3, CH), jnp.int32),
            pltpu.VMEM((4, CH, D), jnp.float32),
            pltpu.VMEM_SHARED((N_AGG, D), jnp.float32),
            pltpu.SemaphoreType.DMA((4,)),
            pltpu.SemaphoreType.DMA((4,)),
            pltpu.SemaphoreType.DMA((4,)),
            pltpu.SemaphoreType.DMA,
        ],
    )
    return k(ab_flat, idx3)


def _s3a_body(x_ref, vt_ref, wux_ref, ne_ref, wue_ref, bu_ref, p_ref):
    oh = (vt_ref[...] ==
          lax.broadcasted_iota(jnp.int32, (1, 8), 1).astype(jnp.float32)
          ).astype(jnp.float32)
    cu = jnp.dot(ne_ref[...], wue_ref[...], preferred_element_type=jnp.float32)
    p_ref[...] = (jnp.dot(x_ref[...], wux_ref[...],
                          preferred_element_type=jnp.float32)
                  + jnp.dot(oh, cu, preferred_element_type=jnp.float32)
                  + bu_ref[...])


def _stage3a(x, vtf, wux, ne8, wue16, bu):
    row = pl.BlockSpec((BN, D), lambda i: (i, 0))
    full = lambda shape: pl.BlockSpec(shape, lambda i: tuple(0 for _ in shape))
    return pl.pallas_call(
        _s3a_body,
        grid=(N_NODES // BN,),
        in_specs=[row, pl.BlockSpec((BN, 1), lambda i: (i, 0)),
                  full((D, D)), full((8, 16)), full((16, D)), full((1, D))],
        out_specs=row,
        out_shape=jax.ShapeDtypeStruct((N_NODES, D), jnp.float32),
    )(x, vtf, wux, ne8, wue16, bu)


def _s3b_body(p_ref, agg_ref, wua_ref, wo_ref, o_ref):
    agg = agg_ref[0] + agg_ref[1]
    u = jnp.maximum(
        p_ref[...] + jnp.dot(agg, wua_ref[...],
                             preferred_element_type=jnp.float32), 0.0)
    o_ref[...] = jnp.dot(u, wo_ref[...], preferred_element_type=jnp.float32)


def _stage3b(p, aggp, wua, wo_pad):
    row = pl.BlockSpec((BN, D), lambda i: (i, 0))
    full = lambda shape: pl.BlockSpec(shape, lambda i: tuple(0 for _ in shape))
    return pl.pallas_call(
        _s3b_body,
        grid=(N_NODES // BN,),
        in_specs=[row, pl.BlockSpec((NC, BN, D), lambda i: (0, i, 0)),
                  full((D, D)), full((D, D))],
        out_specs=row,
        out_shape=jax.ShapeDtypeStruct((N_NODES, D), jnp.float32),
    )(p, aggp, wua, wo_pad)


def kernel(x, edge_index, vertex_type, node_emb, W_msg, b_msg, W_upd, b_upd,
           W_out):
    f32 = jnp.float32
    wax = W_msg[0:128]
    wae16 = jnp.zeros((16, D), f32).at[0:9].set(W_msg[128:137])
    wbx = W_msg[137:265]
    wbe16 = jnp.zeros((16, D), f32).at[0:9].set(W_msg[265:274])
    wux = W_upd[0:128]
    wue16 = jnp.zeros((16, D), f32).at[0:9].set(W_upd[128:137])
    wua = W_upd[137:265]
    ne8 = jnp.zeros((8, 16), f32).at[0:4, 0:9].set(node_emb)
    wo_pad = jnp.zeros((D, D), f32).at[:, 0:3].set(W_out)
    bm = (b_msg.astype(f32)).reshape(1, D)
    bu = (b_upd.astype(f32)).reshape(1, D)

    vtf = vertex_type.astype(f32).reshape(N_NODES, 1)
    src = edge_index[0].reshape(NCHUNK, 1, CH)
    dst = edge_index[1].reshape(NCHUNK, 1, CH)
    idx3 = jnp.concatenate([src, dst + N_NODES, dst],
                           axis=1).reshape(NW, NSB, SB, 3, CH)
    ab = _stage1(x, vtf, wax, wbx, ne8, wae16, wbe16, bm)
    aggp = _stage2(ab.reshape(2 * N_NODES, D), idx3)
    p = _stage3a(x, vtf, wux, ne8, wue16, bu)
    out_full = _stage3b(p, aggp, wua, wo_pad)
    return out_full[:, 0:3]


# speedup vs baseline: 1.0219x; 1.0219x over previous
import jax
import jax.numpy as jnp
from jax import lax
from jax.experimental import pallas as pl
from jax.experimental.pallas import tpu as pltpu
from jax.experimental.pallas import tpu_sc as plsc

N_NODES = 10000
N_EDGES = 320000
D = 128
BN = 1000
CH = 80
NC, NS = 2, 16
NW = NC * NS
NCHUNK = N_EDGES // CH
CPW = NCHUNK // NW
SB = 5
NSB = CPW // SB
N_AGG = 10112
RPT = N_AGG // NS


def _s1_body(x_ref, vt_ref, wax_ref, wbx_ref, ne_ref, wae_ref, wbe_ref,
             bm_ref, ab_ref):
    oh = (vt_ref[...] ==
          lax.broadcasted_iota(jnp.int32, (1, 8), 1).astype(jnp.float32)
          ).astype(jnp.float32)
    ca = jnp.dot(ne_ref[...], wae_ref[...], preferred_element_type=jnp.float32)
    cb = jnp.dot(ne_ref[...], wbe_ref[...], preferred_element_type=jnp.float32)
    xb = x_ref[...]
    ab_ref[0] = (jnp.dot(xb, wax_ref[...], preferred_element_type=jnp.float32)
                 + jnp.dot(oh, ca, preferred_element_type=jnp.float32))
    ab_ref[1] = (jnp.dot(xb, wbx_ref[...], preferred_element_type=jnp.float32)
                 + jnp.dot(oh, cb, preferred_element_type=jnp.float32)
                 + bm_ref[...])


def _stage1(x, vtf, wax, wbx, ne8, wae16, wbe16, bm):
    row = pl.BlockSpec((BN, D), lambda i: (i, 0))
    full = lambda shape: pl.BlockSpec(shape, lambda i: tuple(0 for _ in shape))
    return pl.pallas_call(
        _s1_body,
        grid=(N_NODES // BN,),
        in_specs=[row, pl.BlockSpec((BN, 1), lambda i: (i, 0)),
                  full((D, D)), full((D, D)), full((8, 16)),
                  full((16, D)), full((16, D)), full((1, D))],
        out_specs=pl.BlockSpec((2, BN, D), lambda i: (0, i, 0)),
        out_shape=jax.ShapeDtypeStruct((2, N_NODES, D), jnp.float32),
    )(x, vtf, wax, wbx, ne8, wae16, wbe16, bm)


def _sc_edge_body(ab_hbm, idx_hbm, out_hbm,
                  idx_v, ab_v, agg_sh, sem_g, sem_b, sem_s, sem_i):
    c = lax.axis_index("c")
    s = lax.axis_index("s")
    w = s * NC + c
    row0 = s * RPT
    @plsc.parallel_loop(0, CH, unroll=8)
    def zero_body(r):
        for g in range(8):
            ab_v[0, r, pl.ds(g * 16, 16)] = jnp.zeros((16,), jnp.float32)

    for k in range(7):
        pltpu.sync_copy(ab_v.at[0, pl.ds(0, 80)],
                        agg_sh.at[pl.ds(row0 + k * 80, 80)])
    pltpu.sync_copy(ab_v.at[0, pl.ds(0, 72)],
                    agg_sh.at[pl.ds(row0 + 560, 72)])
    plsc.subcore_barrier()

    def drain_scatter(slot):
        pltpu.make_async_copy(ab_v.at[slot],
                              agg_sh.at[idx_v.at[0, 0, 2]],
                              sem_s.at[slot]).wait()

    def blk_j(i):
        return lax.rem(i // SB, 2), lax.rem(i, SB)

    def start_a(i):
        slot = lax.rem(i, 4)
        blk, j = blk_j(i)

        @pl.when(i >= 4)
        def _():
            drain_scatter(slot)

        pltpu.async_copy(ab_hbm.at[idx_v.at[blk, j, 0]], ab_v.at[slot],
                         sem_g.at[slot])

    def start_b(i):
        slot = lax.rem(i, 4)
        blk, j = blk_j(i)
        pltpu.make_async_copy(ab_hbm.at[idx_v.at[blk, j, 0]], ab_v.at[slot],
                              sem_g.at[slot]).wait()
        pltpu.async_copy(ab_hbm.at[idx_v.at[blk, j, 1]], ab_v.at[slot],
                         sem_b.at[slot], add=True)

    pltpu.sync_copy(idx_hbm.at[w, 0], idx_v.at[0])
    start_a(0)
    start_a(1)
    start_b(0)

    def chunk_body(i, carry):
        slot = lax.rem(i, 4)
        blk, j = blk_j(i)

        @pl.when((j == SB - 3) & (i + 3 < CPW))
        def _():
            pltpu.async_copy(idx_hbm.at[w, i // SB + 1],
                             idx_v.at[1 - blk], sem_i)

        @pl.when((j == SB - 2) & (i + 2 < CPW))
        def _():
            pltpu.make_async_copy(idx_hbm.at[w, 0], idx_v.at[1 - blk],
                                  sem_i).wait()

        @pl.when(i + 2 < CPW)
        def _():
            start_a(i + 2)

        @pl.when(i + 1 < CPW)
        def _():
            start_b(i + 1)

        pltpu.make_async_copy(ab_hbm.at[idx_v.at[blk, j, 1]], ab_v.at[slot],
                              sem_b.at[slot]).wait()

        @plsc.parallel_loop(0, CH, unroll=8)
        def row_body(r):
            for g in range(8):
                sl = pl.ds(g * 16, 16)
                ab_v[slot, r, sl] = jnp.maximum(ab_v[slot, r, sl], 0.0)

        pltpu.async_copy(ab_v.at[slot], agg_sh.at[idx_v.at[blk, j, 2]],
                         sem_s.at[slot], add=True)
        return carry

    lax.fori_loop(0, CPW, chunk_body, 0)
    drain_scatter(0)
    drain_scatter(1)
    drain_scatter(2)
    drain_scatter(3)
    plsc.subcore_barrier()
    pltpu.sync_copy(agg_sh.at[pl.ds(row0, RPT)],
                    out_hbm.at[c, pl.ds(row0, RPT)])


def _stage2(ab_flat, idx3):
    mesh = plsc.VectorSubcoreMesh(core_axis_name="c", subcore_axis_name="s")
    k = pl.kernel(
        _sc_edge_body,
        out_type=jax.ShapeDtypeStruct((NC, N_AGG, D), jnp.float32),
        mesh=mesh,
        scratch_types=[
            pltpu.VMEM((2, SB, 3, CH), jnp.int32),
            pltpu.VMEM((4, CH, D), jnp.float32),
            pltpu.VMEM_SHARED((N_AGG, D), jnp.float32),
            pltpu.SemaphoreType.DMA((4,)),
            pltpu.SemaphoreType.DMA((4,)),
            pltpu.SemaphoreType.DMA((4,)),
            pltpu.SemaphoreType.DMA,
        ],
    )
    return k(ab_flat, idx3)


def _s3a_body(x_ref, vt_ref, wux_ref, ne_ref, wue_ref, bu_ref, p_ref):
    oh = (vt_ref[...] ==
          lax.broadcasted_iota(jnp.int32, (1, 8), 1).astype(jnp.float32)
          ).astype(jnp.float32)
    cu = jnp.dot(ne_ref[...], wue_ref[...], preferred_element_type=jnp.float32)
    p_ref[...] = (jnp.dot(x_ref[...], wux_ref[...],
                          preferred_element_type=jnp.float32)
                  + jnp.dot(oh, cu, preferred_element_type=jnp.float32)
                  + bu_ref[...])


def _stage3a(x, vtf, wux, ne8, wue16, bu):
    row = pl.BlockSpec((BN, D), lambda i: (i, 0))
    full = lambda shape: pl.BlockSpec(shape, lambda i: tuple(0 for _ in shape))
    return pl.pallas_call(
        _s3a_body,
        grid=(N_NODES // BN,),
        in_specs=[row, pl.BlockSpec((BN, 1), lambda i: (i, 0)),
                  full((D, D)), full((8, 16)), full((16, D)), full((1, D))],
        out_specs=row,
        out_shape=jax.ShapeDtypeStruct((N_NODES, D), jnp.float32),
    )(x, vtf, wux, ne8, wue16, bu)


def _s3b_body(p_ref, agg_ref, wua_ref, wo_ref, o_ref):
    agg = agg_ref[0] + agg_ref[1]
    u = jnp.maximum(
        p_ref[...] + jnp.dot(agg, wua_ref[...],
                             preferred_element_type=jnp.float32), 0.0)
    o_ref[...] = jnp.dot(u, wo_ref[...], preferred_element_type=jnp.float32)


def _stage3b(p, aggp, wua, wo_pad):
    row = pl.BlockSpec((BN, D), lambda i: (i, 0))
    full = lambda shape: pl.BlockSpec(shape, lambda i: tuple(0 for _ in shape))
    return pl.pallas_call(
        _s3b_body,
        grid=(N_NODES // BN,),
        in_specs=[row, pl.BlockSpec((NC, BN, D), lambda i: (0, i, 0)),
                  full((D, D)), full((D, D))],
        out_specs=row,
        out_shape=jax.ShapeDtypeStruct((N_NODES, D), jnp.float32),
    )(p, aggp, wua, wo_pad)


def kernel(x, edge_index, vertex_type, node_emb, W_msg, b_msg, W_upd, b_upd,
           W_out):
    f32 = jnp.float32
    wax = W_msg[0:128]
    wae16 = jnp.zeros((16, D), f32).at[0:9].set(W_msg[128:137])
    wbx = W_msg[137:265]
    wbe16 = jnp.zeros((16, D), f32).at[0:9].set(W_msg[265:274])
    wux = W_upd[0:128]
    wue16 = jnp.zeros((16, D), f32).at[0:9].set(W_upd[128:137])
    wua = W_upd[137:265]
    ne8 = jnp.zeros((8, 16), f32).at[0:4, 0:9].set(node_emb)
    wo_pad = jnp.zeros((D, D), f32).at[:, 0:3].set(W_out)
    bm = (b_msg.astype(f32)).reshape(1, D)
    bu = (b_upd.astype(f32)).reshape(1, D)

    vtf = vertex_type.astype(f32).reshape(N_NODES, 1)
    src = edge_index[0].reshape(NCHUNK, 1, CH)
    dst = edge_index[1].reshape(NCHUNK, 1, CH)
    idx3 = jnp.concatenate([src, dst + N_NODES, dst],
                           axis=1).reshape(NW, NSB, SB, 3, CH)
    ab = _stage1(x, vtf, wax, wbx, ne8, wae16, wbe16, bm)
    aggp = _stage2(ab.reshape(2 * N_NODES, D), idx3)
    p = _stage3a(x, vtf, wux, ne8, wue16, bu)
    out_full = _stage3b(p, aggp, wua, wo_pad)
    return out_full[:, 0:3]
